# Initial kernel scaffold; baseline (speedup 1.0000x reference)
import jax
import jax.numpy as jnp
from jax.experimental import pallas as pl


def kernel(router_weights, n_routed_experts, num_experts_per_tok):
    raise NotImplementedError("write your pallas kernel here")


# speedup vs baseline: 2.3023x; 2.3023x over previous
import functools

import jax
import jax.numpy as jnp
from jax import lax
from jax.experimental import pallas as pl
from jax.experimental.pallas import tpu as pltpu
from jax.experimental.pallas import tpu_sc as plsc

_LOSS_WEIGHT = 0.01
_L = 16
_NUM_CORES = 2
_NUM_SUBCORES = 16
_CHUNK = 1024


def _sc_body(num_tokens, num_experts, rw_hbm, out_hbm, buf, thr, part):
    cid = lax.axis_index("c")
    sid = lax.axis_index("s")
    wid = sid * _NUM_CORES + cid
    lane_iota = lax.broadcasted_iota(jnp.int32, (_L,), 0)
    neg_inf = jnp.full((_L,), -jnp.inf, dtype=jnp.float32)
    zeros = jnp.zeros((_L,), dtype=jnp.float32)
    n_groups = num_experts // _L

    base_idx = lane_iota * num_experts

    def chunk_step(c, accs):
        pltpu.sync_copy(
            rw_hbm.at[wid, pl.ds(c * _CHUNK * num_experts, _CHUNK * num_experts)],
            buf)

        def topk_step(g, carry):
            idx0 = base_idx + g * (_L * num_experts)
            m1 = neg_inf
            m2 = neg_inf
            for e in range(num_experts):
                v = plsc.load_gather(buf, [idx0 + e])
                m2 = jnp.maximum(m2, jnp.minimum(v, m1))
                m1 = jnp.maximum(m1, v)
            thr[pl.ds(g * _L, _L)] = m2
            return carry

        lax.fori_loop(0, _CHUNK // _L, topk_step, 0)

        def acc_step(g, carry):
            sums, cnts = carry
            thv = thr[pl.ds(g * _L, _L)]
            sums = list(sums)
            cnts = list(cnts)
            for l in range(_L):
                t = g * _L + l
                th = jnp.full((_L,), thv[l])
                for j in range(n_groups):
                    v = buf[pl.ds(t * num_experts + j * _L, _L)]
                    sums[j] = sums[j] + v
                    cnts[j] = cnts[j] + jnp.where(v >= th, 1.0, 0.0)
            return (tuple(sums), tuple(cnts))

        return lax.fori_loop(0, _CHUNK // _L, acc_step, accs)

    init = ((zeros,) * n_groups, (zeros,) * n_groups)
    sums, cnts = lax.fori_loop(0, num_tokens // _CHUNK, chunk_step, init)

    acc = zeros
    for j in range(n_groups):
        acc = acc + sums[j] * cnts[j]
    part[...] = acc
    pltpu.sync_copy(part, out_hbm.at[wid])


def kernel(router_weights, n_routed_experts, num_experts_per_tok):
    num_layers, num_tokens, num_experts = router_weights.shape
    rw = router_weights.astype(jnp.float32).reshape(num_layers, num_tokens * num_experts)
    num_workers = _NUM_CORES * _NUM_SUBCORES
    assert num_layers == num_workers and num_experts % _L == 0
    assert num_tokens % _CHUNK == 0

    run = pl.kernel(
        functools.partial(_sc_body, num_tokens, num_experts),
        out_type=jax.ShapeDtypeStruct((num_workers, _L), jnp.float32),
        mesh=plsc.VectorSubcoreMesh(core_axis_name="c", subcore_axis_name="s"),
        scratch_types=[
            pltpu.VMEM((_CHUNK * num_experts,), jnp.float32),
            pltpu.VMEM((_CHUNK,), jnp.float32),
            pltpu.VMEM((_L,), jnp.float32),
        ],
        compiler_params=pltpu.CompilerParams(needs_layout_passes=False),
    )
    partials = run(rw)
    scale = n_routed_experts / (num_tokens * num_experts_per_tok)
    return partials.sum() * jnp.float32(scale / num_tokens * _LOSS_WEIGHT)


# speedup vs baseline: 5.2170x; 2.2660x over previous
import functools

import jax
import jax.numpy as jnp
from jax import lax
from jax.experimental import pallas as pl
from jax.experimental.pallas import tpu as pltpu
from jax.experimental.pallas import tpu_sc as plsc

_LOSS_WEIGHT = 0.01
_L = 16
_NUM_CORES = 2
_NUM_SUBCORES = 16
_CHUNK = 1024


def _sc_body(num_tokens, num_experts, rw_hbm, out_hbm, buf, part):
    cid = lax.axis_index("c")
    sid = lax.axis_index("s")
    wid = sid * _NUM_CORES + cid
    zeros = jnp.zeros((_L,), dtype=jnp.float32)
    n_groups = num_experts // _L

    def chunk_step(c, accs):
        pltpu.sync_copy(
            rw_hbm.at[wid, pl.ds(c * _CHUNK * num_experts, _CHUNK * num_experts)],
            buf)

        def token_step(t, carry):
            sums, cnts = carry
            v = [buf[pl.ds(t * num_experts + j * _L, _L)]
                 for j in range(n_groups)]
            a = jnp.maximum(v[0], v[1])
            b = jnp.minimum(v[0], v[1])
            c2 = jnp.maximum(v[2], v[3])
            d = jnp.minimum(v[2], v[3])
            m1 = jnp.maximum(a, c2)
            m2 = jnp.maximum(jnp.minimum(a, c2), jnp.maximum(b, d))
            sk, sv = plsc.sort_key_val(m1, m2, descending=True)
            th = jnp.full((_L,), jnp.maximum(sk[1], sv[0]))
            sums = tuple(sums[j] + v[j] for j in range(n_groups))
            cnts = tuple(
                cnts[j] + jnp.where(v[j] >= th, 1.0, 0.0)
                for j in range(n_groups))
            return (sums, cnts)

        return lax.fori_loop(0, _CHUNK, token_step, accs, unroll=8)

    init = ((zeros,) * n_groups, (zeros,) * n_groups)
    sums, cnts = lax.fori_loop(0, num_tokens // _CHUNK, chunk_step, init)

    acc = zeros
    for j in range(n_groups):
        acc = acc + sums[j] * cnts[j]
    part[...] = acc
    pltpu.sync_copy(part, out_hbm.at[wid])


def kernel(router_weights, n_routed_experts, num_experts_per_tok):
    num_layers, num_tokens, num_experts = router_weights.shape
    rw = router_weights.astype(jnp.float32).reshape(
        num_layers, num_tokens * num_experts)
    num_workers = _NUM_CORES * _NUM_SUBCORES
    assert num_layers == num_workers and num_experts == 4 * _L
    assert num_tokens % _CHUNK == 0

    run = pl.kernel(
        functools.partial(_sc_body, num_tokens, num_experts),
        out_type=jax.ShapeDtypeStruct((num_workers, _L), jnp.float32),
        mesh=plsc.VectorSubcoreMesh(core_axis_name="c", subcore_axis_name="s"),
        scratch_types=[
            pltpu.VMEM((_CHUNK * num_experts,), jnp.float32),
            pltpu.VMEM((_L,), jnp.float32),
        ],
        compiler_params=pltpu.CompilerParams(needs_layout_passes=False),
    )
    partials = run(rw)
    scale = n_routed_experts / (num_tokens * num_experts_per_tok)
    return partials.sum() * jnp.float32(scale / num_tokens * _LOSS_WEIGHT)
